# Initial kernel scaffold; baseline (speedup 1.0000x reference)
import functools

import jax
import jax.numpy as jnp
from jax import lax
from jax.experimental import pallas as pl
from jax.experimental.pallas import tpu as pltpu
from jax.experimental.pallas import tpu_sc as plsc

TILES = 16
LANES = 16


def _fill_1d(ref, n, value):
    v = jnp.full((LANES,), value, jnp.float32)

    def body(i, _):
        ref[pl.ds(i * LANES, LANES)] = v
        return 0

    lax.fori_loop(0, n // LANES, body, 0)


def _fill_2d(ref, rows, cols, value):
    v = jnp.full((LANES,), value, jnp.float32)
    ncol = cols // LANES

    def body(r, _):
        for c in range(ncol):
            ref[r, pl.ds(c * LANES, LANES)] = v
        return 0

    lax.fori_loop(0, rows, body, 0)


def _node_sc(x, lg2d, N, E, D):
    NP = -(-N // 128) * 128
    SL = NP // TILES
    B = 200
    assert E % (TILES * B) == 0 and SL % 128 == 0
    NB = E // (TILES * B)
    ZR = 128

    mesh = plsc.VectorSubcoreMesh(core_axis_name="c", subcore_axis_name="s")

    @functools.partial(
        pl.kernel,
        out_type=(
            jax.ShapeDtypeStruct((2, NP, D), jnp.float32),
            jax.ShapeDtypeStruct((2, NP), jnp.float32),
        ),
        mesh=mesh,
        scratch_types=[
            pltpu.VMEM((B, D), jnp.float32),
            pltpu.VMEM((2, B // 2), jnp.int32),
            pltpu.VMEM((128,), jnp.float32),
            pltpu.VMEM((ZR, D), jnp.float32),
            pltpu.VMEM((SL,), jnp.float32),
            pltpu.VMEM_SHARED((NP, D), jnp.float32),
            pltpu.VMEM_SHARED((NP,), jnp.float32),
        ],
    )
    def k(x_ref, lg_ref, acc_out, cnt_out, buf, idx, ones, zb2, zb1, acc_sh,
          cnt_sh):
        cid = lax.axis_index("c")
        sid = lax.axis_index("s")
        col0 = jnp.where(cid == 0, D, 0)

        _fill_1d(ones, 128, 1.0)
        _fill_2d(zb2, ZR, D, 0.0)
        _fill_1d(zb1, SL, 0.0)

        for i in range(SL // ZR):
            pltpu.sync_copy(zb2, acc_sh.at[pl.ds(sid * SL + i * ZR, ZR), :])
        pltpu.sync_copy(zb1, cnt_sh.at[pl.ds(sid * SL, SL)])
        plsc.subcore_barrier()

        def block(kk, _):
            base = sid * (E // TILES) + kk * B
            irow = sid * (E // TILES // (B // 2)) + kk * 2
            pltpu.sync_copy(x_ref.at[pl.ds(base, B), pl.ds(col0, D)], buf)
            pltpu.sync_copy(lg_ref.at[cid, pl.ds(irow, 2), :], idx)
            for j in range(2):
                pltpu.sync_copy(buf.at[pl.ds(j * (B // 2), B // 2), :],
                                acc_sh.at[idx.at[j]], add=True)
                pltpu.sync_copy(ones.at[pl.ds(0, B // 2)],
                                cnt_sh.at[idx.at[j]], add=True)
            return 0

        lax.fori_loop(0, NB, block, 0)
        plsc.subcore_barrier()

        pltpu.sync_copy(acc_sh.at[pl.ds(sid * SL, SL), :],
                        acc_out.at[cid, pl.ds(sid * SL, SL), :])
        pltpu.sync_copy(cnt_sh.at[pl.ds(sid * SL, SL)],
                        cnt_out.at[cid, pl.ds(sid * SL, SL)])

    return k(x, lg2d)


def _edge_sc(ea, ei3d, E, ELG, DE):
    B = 640
    assert ELG % (TILES * B) == 0
    NB = ELG // (TILES * B)
    PT = ELG // TILES
    CH = -(-E // 3 // 128) * 128
    sizes = [CH, CH, E - 2 * CH]
    assert all(s > 0 and s % (TILES * 8) == 0 for s in sizes)
    TR = 64
    AR = CH + TR
    ZR = min(836, max(s // TILES for s in sizes))

    mesh = plsc.VectorSubcoreMesh(core_axis_name="c", subcore_axis_name="s")

    @functools.partial(
        pl.kernel,
        out_type=(
            jax.ShapeDtypeStruct((2, E, DE), jnp.float32),
            jax.ShapeDtypeStruct((2, E), jnp.float32),
        ),
        mesh=mesh,
        scratch_types=[
            pltpu.VMEM((B, DE), jnp.float32),
            pltpu.VMEM((B // 128, 128), jnp.int32),
            pltpu.VMEM((B // 128, 128), jnp.int32),
            pltpu.VMEM((128,), jnp.float32),
            pltpu.VMEM((ZR, DE), jnp.float32),
            pltpu.VMEM((ZR,), jnp.float32),
            pltpu.VMEM_SHARED((AR, DE), jnp.float32),
            pltpu.VMEM_SHARED((AR,), jnp.float32),
        ],
    )
    def k(ea_ref, ei_ref, acc_out, cnt_out, buf, idx, idxp, ones, zb2, zb1,
          acc_sh, cnt_sh):
        cid = lax.axis_index("c")
        sid = lax.axis_index("s")
        col0 = jnp.where(cid == 0, DE, 0)
        iota16 = lax.iota(jnp.int32, LANES)

        _fill_1d(ones, 128, 1.0)
        _fill_2d(zb2, ZR, DE, 0.0)
        _fill_1d(zb1, ZR, 0.0)

        for c, size in enumerate(sizes):
            lo = c * CH
            sl = size // TILES
            nfull, tail = divmod(sl, ZR)
            for i in range(nfull):
                pltpu.sync_copy(zb2, acc_sh.at[pl.ds(sid * sl + i * ZR, ZR), :])
                pltpu.sync_copy(zb1, cnt_sh.at[pl.ds(sid * sl + i * ZR, ZR)])
            if tail:
                pltpu.sync_copy(zb2.at[pl.ds(0, tail), :],
                                acc_sh.at[pl.ds(sid * sl + nfull * ZR, tail), :])
                pltpu.sync_copy(zb1.at[pl.ds(0, tail)],
                                cnt_sh.at[pl.ds(sid * sl + nfull * ZR, tail)])

            @pl.when(sid == 0)
            def _():
                pltpu.sync_copy(zb2.at[pl.ds(0, TR), :],
                                acc_sh.at[pl.ds(CH, TR), :])
                pltpu.sync_copy(zb1.at[pl.ds(0, TR)],
                                cnt_sh.at[pl.ds(CH, TR)])

            plsc.subcore_barrier()

            def block(kk, _):
                base = sid * PT + kk * B
                irow = sid * (PT // 128) + kk * (B // 128)
                pltpu.sync_copy(ea_ref.at[pl.ds(base, B), pl.ds(col0, DE)], buf)
                pltpu.sync_copy(ei_ref.at[cid, pl.ds(irow, B // 128), :], idx)
                for r in range(B // 128):
                    for l in range(8):
                        v = idx[r, pl.ds(l * LANES, LANES)]
                        ok = (v >= lo) & (v < lo + size)
                        trash = CH + ((r * 8 + l) % 4) * LANES + iota16
                        idxp[r, pl.ds(l * LANES, LANES)] = jnp.where(
                            ok, v - lo, trash)
                for r in range(B // 128):
                    pltpu.sync_copy(buf.at[pl.ds(r * 128, 128), :],
                                    acc_sh.at[idxp.at[r]], add=True)
                    pltpu.sync_copy(ones, cnt_sh.at[idxp.at[r]], add=True)
                return 0

            lax.fori_loop(0, NB, block, 0)
            plsc.subcore_barrier()

            pltpu.sync_copy(acc_sh.at[pl.ds(sid * sl, sl), :],
                            acc_out.at[cid, pl.ds(lo + sid * sl, sl), :])
            pltpu.sync_copy(cnt_sh.at[pl.ds(sid * sl, sl)],
                            cnt_out.at[cid, pl.ds(lo + sid * sl, sl)])

    return k(ea, ei3d)


def _finalize(acc, cnt, rows, dim, blk):
    def body(acc_ref, cnt_ref, out_ref):
        ra = 1.0 / jnp.maximum(cnt_ref[0], 1.0)
        rb = 1.0 / jnp.maximum(cnt_ref[1], 1.0)
        out_ref[...] = acc_ref[0] * ra + acc_ref[1] * rb

    assert rows % blk == 0
    cnt3 = cnt[:, :, None]
    return pl.pallas_call(
        body,
        out_shape=jax.ShapeDtypeStruct((rows, dim), jnp.float32),
        grid=(rows // blk,),
        in_specs=[
            pl.BlockSpec((2, blk, dim), lambda i: (0, i, 0)),
            pl.BlockSpec((2, blk, 1), lambda i: (0, i, 0)),
        ],
        out_specs=pl.BlockSpec((blk, dim), lambda i: (i, 0)),
    )(acc, cnt3)


def kernel(x, lg_node_idx, edge_attr, edge_index, org_edge_attr, org_x,
           org_edge_index):
    N, D = org_x.shape
    E, DE = org_edge_attr.shape
    ELG = edge_attr.shape[0]

    lg2d = lg_node_idx.T.reshape(2, E // 100, 100)
    ei3d = edge_index.reshape(2, ELG // 128, 128)

    nacc, ncnt = _node_sc(x, lg2d, N, E, D)
    eacc, ecnt = _edge_sc(edge_attr, ei3d, E, ELG, DE)

    new_x = _finalize(nacc, ncnt, N, D, 1000 if N % 1000 == 0 else N)
    new_edge_attr = _finalize(eacc, ecnt, E, DE, 4000 if E % 4000 == 0 else E)
    return new_x, new_edge_attr


# speedup vs baseline: 4.4926x; 4.4926x over previous
import functools

import jax
import jax.numpy as jnp
from jax import lax
from jax.experimental import pallas as pl
from jax.experimental.pallas import tpu as pltpu
from jax.experimental.pallas import tpu_sc as plsc

TILES = 16
LANES = 16


def _fill_1d(ref, n, value):
    v = jnp.full((LANES,), value, jnp.float32)

    def body(i, _):
        ref[pl.ds(i * LANES, LANES)] = v
        return 0

    lax.fori_loop(0, n // LANES, body, 0)


def _fill_2d(ref, rows, cols, value):
    v = jnp.full((LANES,), value, jnp.float32)
    ncol = cols // LANES

    def body(r, _):
        for c in range(ncol):
            ref[r, pl.ds(c * LANES, LANES)] = v
        return 0

    lax.fori_loop(0, rows, body, 0)


def _scale_rows(buf, cbuf, off, rows, dim):
    nseg = dim // LANES
    ngrp, rem = divmod(rows, LANES)

    def group(g, nrow):
        cv = cbuf[pl.ds(off + g * LANES, LANES)]
        rv = 1.0 / jnp.maximum(cv, 1.0)
        for t in range(nrow):
            rinv = rv[t]
            r = g * LANES + t
            for s in range(nseg):
                buf[r, pl.ds(s * LANES, LANES)] = (
                    buf[r, pl.ds(s * LANES, LANES)] * rinv)

    def body(g, _):
        group(g, LANES)
        return 0

    lax.fori_loop(0, ngrp, body, 0)
    if rem:
        group(ngrp, rem)


def _node_sc(x, lg2d, N, E, D):
    NP = -(-N // 2048) * 2048
    SL = NP // TILES
    B = 200
    assert E % (TILES * B) == 0 and SL % 128 == 0
    NB = E // (TILES * B)
    ZR = 128

    mesh = plsc.VectorSubcoreMesh(core_axis_name="c", subcore_axis_name="s",
                                  num_cores=2, num_subcores=TILES)

    @functools.partial(
        pl.kernel,
        out_type=jax.ShapeDtypeStruct((2, NP, D), jnp.float32),
        mesh=mesh,
        scratch_types=[
            pltpu.VMEM((B, D), jnp.float32),
            pltpu.VMEM((2, B // 2), jnp.int32),
            pltpu.VMEM((128,), jnp.float32),
            pltpu.VMEM((ZR, D), jnp.float32),
            pltpu.VMEM((SL,), jnp.float32),
            pltpu.VMEM_SHARED((NP, D), jnp.float32),
            pltpu.VMEM_SHARED((NP,), jnp.float32),
        ],
    )
    def k(x_ref, lg_ref, mean_out, buf, idx, ones, zb2, zb1, acc_sh, cnt_sh):
        cid = lax.axis_index("c")
        sid = lax.axis_index("s")
        col0 = jnp.where(cid == 0, D, 0)

        _fill_1d(ones, 128, 1.0)
        _fill_2d(zb2, ZR, D, 0.0)
        _fill_1d(zb1, SL, 0.0)

        for i in range(SL // ZR):
            pltpu.sync_copy(zb2, acc_sh.at[pl.ds(sid * SL + i * ZR, ZR), :])
        pltpu.sync_copy(zb1, cnt_sh.at[pl.ds(sid * SL, SL)])
        plsc.subcore_barrier()

        def block(kk, _):
            base = sid * (E // TILES) + kk * B
            irow = sid * (E // TILES // (B // 2)) + kk * 2
            pltpu.sync_copy(x_ref.at[pl.ds(base, B), pl.ds(col0, D)], buf)
            pltpu.sync_copy(lg_ref.at[cid, pl.ds(irow, 2), :], idx)
            for j in range(2):
                pltpu.sync_copy(buf.at[pl.ds(j * (B // 2), B // 2), :],
                                acc_sh.at[idx.at[j]], add=True)
                pltpu.sync_copy(ones.at[pl.ds(0, B // 2)],
                                cnt_sh.at[idx.at[j]], add=True)
            return 0

        lax.fori_loop(0, NB, block, 0)
        plsc.subcore_barrier()

        P = 128
        pltpu.sync_copy(cnt_sh.at[pl.ds(sid * SL, SL)], zb1)

        def dump(i, _):
            r0 = sid * SL + i * P
            pltpu.sync_copy(acc_sh.at[pl.ds(r0, P), :], buf.at[pl.ds(0, P), :])
            _scale_rows(buf, zb1, i * P, P, D)
            pltpu.sync_copy(buf.at[pl.ds(0, P), :],
                            mean_out.at[cid, pl.ds(r0, P), :])
            return 0

        lax.fori_loop(0, SL // P, dump, 0)

    return k(x, lg2d)


def _edge_sc(ea, ei3d, E, ELG, DE):
    B = 640
    assert ELG % (TILES * B) == 0
    NB = ELG // (TILES * B)
    PT = ELG // TILES
    NCH = 4
    CH = -(-E // NCH // 128) * 128
    sizes = [CH] * (NCH - 1) + [E - (NCH - 1) * CH]
    assert all(0 < s <= CH and s % (TILES * 8) == 0 for s in sizes)
    TR = 64
    AR = CH + TR
    ZR = min(832, max(s // TILES for s in sizes))

    mesh = plsc.VectorSubcoreMesh(core_axis_name="c", subcore_axis_name="s",
                                  num_cores=2, num_subcores=TILES)

    @functools.partial(
        pl.kernel,
        out_type=jax.ShapeDtypeStruct((2, E, DE), jnp.float32),
        mesh=mesh,
        scratch_types=[
            pltpu.VMEM((B, DE), jnp.float32),
            pltpu.VMEM((B // 128, 128), jnp.int32),
            pltpu.VMEM((B // 128, 128), jnp.int32),
            pltpu.VMEM((128,), jnp.float32),
            pltpu.VMEM((ZR, DE), jnp.float32),
            pltpu.VMEM((ZR,), jnp.float32),
            pltpu.VMEM((B,), jnp.float32),
            pltpu.VMEM_SHARED((AR, DE), jnp.float32),
            pltpu.VMEM_SHARED((AR,), jnp.float32),
        ],
        compiler_params=pltpu.CompilerParams(use_tc_tiling_on_sc=False),
    )
    def k(ea_ref, ei_ref, mean_out, buf, idx, idxp, ones, zb2, zb1, cbuf,
          acc_sh, cnt_sh):
        cid = lax.axis_index("c")
        sid = lax.axis_index("s")
        col0 = jnp.where(cid == 0, DE, 0)
        iota16 = lax.iota(jnp.int32, LANES)

        _fill_1d(ones, 128, 1.0)
        _fill_2d(zb2, ZR, DE, 0.0)
        _fill_1d(zb1, ZR, 0.0)

        for c, size in enumerate(sizes):
            lo = c * CH
            sl = size // TILES
            nfull, tail = divmod(sl, ZR)
            for i in range(nfull):
                pltpu.sync_copy(zb2, acc_sh.at[pl.ds(sid * sl + i * ZR, ZR), :])
                pltpu.sync_copy(zb1, cnt_sh.at[pl.ds(sid * sl + i * ZR, ZR)])
            if tail:
                pltpu.sync_copy(zb2.at[pl.ds(0, tail), :],
                                acc_sh.at[pl.ds(sid * sl + nfull * ZR, tail), :])
                pltpu.sync_copy(zb1.at[pl.ds(0, tail)],
                                cnt_sh.at[pl.ds(sid * sl + nfull * ZR, tail)])

            @pl.when(sid == 0)
            def _():
                pltpu.sync_copy(zb2.at[pl.ds(0, TR), :],
                                acc_sh.at[pl.ds(CH, TR), :])
                pltpu.sync_copy(zb1.at[pl.ds(0, TR)],
                                cnt_sh.at[pl.ds(CH, TR)])

            plsc.subcore_barrier()

            def block(kk, _):
                base = sid * PT + kk * B
                irow = sid * (PT // 128) + kk * (B // 128)
                pltpu.sync_copy(ea_ref.at[pl.ds(base, B), pl.ds(col0, DE)], buf)
                pltpu.sync_copy(ei_ref.at[cid, pl.ds(irow, B // 128), :], idx)
                for r in range(B // 128):
                    for l in range(8):
                        v = idx[r, pl.ds(l * LANES, LANES)]
                        ok = (v >= lo) & (v < lo + size)
                        trash = CH + ((r * 8 + l) % 4) * LANES + iota16
                        idxp[r, pl.ds(l * LANES, LANES)] = jnp.where(
                            ok, v - lo, trash)
                for r in range(B // 128):
                    pltpu.sync_copy(buf.at[pl.ds(r * 128, 128), :],
                                    acc_sh.at[idxp.at[r]], add=True)
                    pltpu.sync_copy(ones, cnt_sh.at[idxp.at[r]], add=True)
                return 0

            lax.fori_loop(0, NB, block, 0)
            plsc.subcore_barrier()

            nsub, dtail = divmod(sl, B)

            def dump_piece(r0, nrows):
                pltpu.sync_copy(acc_sh.at[pl.ds(r0, nrows), :],
                                buf.at[pl.ds(0, nrows), :])
                pltpu.sync_copy(cnt_sh.at[pl.ds(r0, nrows)],
                                cbuf.at[pl.ds(0, nrows)])
                _scale_rows(buf, cbuf, 0, nrows, DE)
                pltpu.sync_copy(buf.at[pl.ds(0, nrows), :],
                                mean_out.at[cid, pl.ds(lo + r0, nrows), :])

            def dump(i, _):
                dump_piece(sid * sl + i * B, B)
                return 0

            lax.fori_loop(0, nsub, dump, 0)
            if dtail:
                dump_piece(sid * sl + nsub * B, dtail)

    return k(ea, ei3d)


def _finalize(mean, rows, dim, blk):
    def body(m_ref, out_ref):
        out_ref[...] = m_ref[0] + m_ref[1]

    assert rows % blk == 0
    return pl.pallas_call(
        body,
        out_shape=jax.ShapeDtypeStruct((rows, dim), jnp.float32),
        grid=(rows // blk,),
        in_specs=[pl.BlockSpec((2, blk, dim), lambda i: (0, i, 0))],
        out_specs=pl.BlockSpec((blk, dim), lambda i: (i, 0)),
    )(mean)


def kernel(x, lg_node_idx, edge_attr, edge_index, org_edge_attr, org_x,
           org_edge_index):
    N, D = org_x.shape
    E, DE = org_edge_attr.shape
    ELG = edge_attr.shape[0]

    lg2d = lg_node_idx.T.reshape(2, E // 100, 100)
    ei3d = edge_index.reshape(2, ELG // 128, 128)

    nmean = _node_sc(x, lg2d, N, E, D)
    emean = _edge_sc(edge_attr, ei3d, E, ELG, DE)

    new_x = _finalize(nmean, N, D, 1000 if N % 1000 == 0 else N)
    WF = 128 // DE
    ew = _finalize(emean.reshape(2, E // WF, WF * DE), E // WF, WF * DE,
                   4000 if (E // WF) % 4000 == 0 else E // WF)
    new_edge_attr = ew.reshape(E, DE)
    return new_x, new_edge_attr


# speedup vs baseline: 6.9324x; 1.5431x over previous
import functools

import jax
import jax.numpy as jnp
from jax import lax
from jax.experimental import pallas as pl
from jax.experimental.pallas import tpu as pltpu
from jax.experimental.pallas import tpu_sc as plsc

TILES = 16
LANES = 16


def _fill_1d(ref, n, value):
    v = jnp.full((LANES,), value, jnp.float32)

    def body(i, _):
        ref[pl.ds(i * LANES, LANES)] = v
        return 0

    lax.fori_loop(0, n // LANES, body, 0)


def _scale_rows(buf, cbuf, off, rows, dim):
    nseg = dim // LANES
    ngrp, rem = divmod(rows, LANES)

    def group(g, nrow):
        cv = cbuf[pl.ds(off + g * LANES, LANES)]
        rv = 1.0 / jnp.maximum(cv, 1.0)
        for t in range(nrow):
            rinv = rv[t]
            r = g * LANES + t
            for s in range(nseg):
                buf[r, pl.ds(s * LANES, LANES)] = (
                    buf[r, pl.ds(s * LANES, LANES)] * rinv)

    def body(g, _):
        group(g, LANES)
        return 0

    lax.fori_loop(0, ngrp, body, 0)
    if rem:
        group(ngrp, rem)


def _node_sc(x, lg2d, z2, z1, N, E, D):
    NP = -(-N // 2048) * 2048
    SL = NP // TILES
    B = 200
    assert E % (TILES * B) == 0 and SL % 128 == 0
    NB = E // (TILES * B)
    NG = B // 100

    mesh = plsc.VectorSubcoreMesh(core_axis_name="c", subcore_axis_name="s",
                                  num_cores=2, num_subcores=TILES)

    @functools.partial(
        pl.kernel,
        out_type=jax.ShapeDtypeStruct((2, NP, D), jnp.float32),
        mesh=mesh,
        scratch_types=[
            pltpu.VMEM((B, D), jnp.float32),
            pltpu.VMEM((NG, 100), jnp.int32),
            pltpu.VMEM((128,), jnp.float32),
            pltpu.VMEM((SL,), jnp.float32),
            pltpu.SemaphoreType.DMA,
            pltpu.SemaphoreType.DMA,
            pltpu.VMEM_SHARED((NP, D), jnp.float32),
            pltpu.VMEM_SHARED((NP,), jnp.float32),
        ],
    )
    def k(x_ref, lg_ref, z2_ref, z1_ref, mean_out, buf, idx, ones, cbuf,
          gsem, ssem, acc_sh, cnt_sh):
        cid = lax.axis_index("c")
        sid = lax.axis_index("s")
        col0 = jnp.where(cid == 0, D, 0)

        _fill_1d(ones, 128, 1.0)

        for i in range(SL // 128):
            pltpu.sync_copy(z2_ref.at[pl.ds(0, 128), :],
                            acc_sh.at[pl.ds(sid * SL + i * 128, 128), :])
        pltpu.sync_copy(z1_ref.at[pl.ds(0, SL)], cnt_sh.at[pl.ds(sid * SL, SL)])
        plsc.subcore_barrier()

        def block(kk, _):
            base = sid * (E // TILES) + kk * B
            irow = sid * (E // TILES // 100) + kk * NG
            g1 = pltpu.async_copy(x_ref.at[pl.ds(base, B), pl.ds(col0, D)],
                                  buf, gsem)
            g2 = pltpu.async_copy(lg_ref.at[cid, pl.ds(irow, NG), :], idx,
                                  gsem)
            g1.wait()
            g2.wait()
            descs = []
            for j in range(NG):
                descs.append(pltpu.async_copy(
                    buf.at[pl.ds(j * 100, 100), :],
                    acc_sh.at[idx.at[j]], ssem, add=True))
                descs.append(pltpu.async_copy(
                    ones.at[pl.ds(0, 100)],
                    cnt_sh.at[idx.at[j]], ssem, add=True))
            for d in descs:
                d.wait()
            return 0

        lax.fori_loop(0, NB, block, 0)
        plsc.subcore_barrier()

        P = 128
        pltpu.sync_copy(cnt_sh.at[pl.ds(sid * SL, SL)], cbuf)

        def dump(i, _):
            r0 = sid * SL + i * P
            pltpu.sync_copy(acc_sh.at[pl.ds(r0, P), :], buf.at[pl.ds(0, P), :])
            _scale_rows(buf, cbuf, i * P, P, D)
            pltpu.sync_copy(buf.at[pl.ds(0, P), :],
                            mean_out.at[cid, pl.ds(r0, P), :])
            return 0

        lax.fori_loop(0, SL // P, dump, 0)

    return k(x, lg2d, z2, z1)


def _edge_sc(ea, ei3d, z2, z1, E, ELG, DE):
    B = 640
    NR = B // 128
    assert ELG % (TILES * B) == 0
    NB = ELG // (TILES * B)
    PT = ELG // TILES
    NCH = 4
    CH = -(-E // NCH // 128) * 128
    sizes = [CH] * (NCH - 1) + [E - (NCH - 1) * CH]
    assert all(0 < s <= CH and s % (TILES * 8) == 0 for s in sizes)
    TR = 64
    AR = CH + TR
    ZR = 832

    mesh = plsc.VectorSubcoreMesh(core_axis_name="c", subcore_axis_name="s",
                                  num_cores=2, num_subcores=TILES)

    @functools.partial(
        pl.kernel,
        out_type=jax.ShapeDtypeStruct((2, E, DE), jnp.float32),
        mesh=mesh,
        scratch_types=[
            pltpu.VMEM((B, DE), jnp.float32),
            pltpu.VMEM((B, DE), jnp.float32),
            pltpu.VMEM((NR, 128), jnp.int32),
            pltpu.VMEM((NR, 128), jnp.int32),
            pltpu.VMEM((NR, 128), jnp.int32),
            pltpu.VMEM((NR, 128), jnp.int32),
            pltpu.VMEM((128,), jnp.float32),
            pltpu.VMEM((B,), jnp.float32),
            pltpu.SemaphoreType.DMA,
            pltpu.SemaphoreType.DMA,
            pltpu.SemaphoreType.DMA,
            pltpu.VMEM_SHARED((AR, DE), jnp.float32),
            pltpu.VMEM_SHARED((AR,), jnp.float32),
        ],
        compiler_params=pltpu.CompilerParams(use_tc_tiling_on_sc=False),
    )
    def k(ea_ref, ei_ref, z2_ref, z1_ref, mean_out, buf0, buf1, idx0, idx1,
          idxp0, idxp1, ones, cbuf, gsem0, gsem1, ssem, acc_sh, cnt_sh):
        cid = lax.axis_index("c")
        sid = lax.axis_index("s")
        col0 = jnp.where(cid == 0, DE, 0)
        iota16 = lax.iota(jnp.int32, LANES)
        bufs = (buf0, buf1)
        idxs = (idx0, idx1)
        idxps = (idxp0, idxp1)
        gsems = (gsem0, gsem1)

        _fill_1d(ones, 128, 1.0)

        def fire_gather(kk, slot):
            base = sid * PT + kk * B
            irow = sid * (PT // 128) + kk * NR
            pltpu.async_copy(ea_ref.at[pl.ds(base, B), pl.ds(col0, DE)],
                             bufs[slot], gsems[slot])
            pltpu.async_copy(ei_ref.at[cid, pl.ds(irow, NR), :], idxs[slot],
                             gsems[slot])

        def wait_gather(kk, slot):
            base = sid * PT + kk * B
            irow = sid * (PT // 128) + kk * NR
            pltpu.make_async_copy(
                ea_ref.at[pl.ds(base, B), pl.ds(col0, DE)], bufs[slot],
                gsems[slot]).wait()
            pltpu.make_async_copy(
                ei_ref.at[cid, pl.ds(irow, NR), :], idxs[slot],
                gsems[slot]).wait()

        for c, size in enumerate(sizes):
            lo = c * CH
            sl = size // TILES
            nfull, tail = divmod(sl, ZR)
            for i in range(nfull):
                pltpu.sync_copy(z2_ref.at[pl.ds(0, ZR), pl.ds(0, DE)],
                                acc_sh.at[pl.ds(sid * sl + i * ZR, ZR), :])
            if tail:
                pltpu.sync_copy(z2_ref.at[pl.ds(0, tail), pl.ds(0, DE)],
                                acc_sh.at[pl.ds(sid * sl + nfull * ZR, tail), :])
            pltpu.sync_copy(z1_ref.at[pl.ds(0, sl)],
                            cnt_sh.at[pl.ds(sid * sl, sl)])

            @pl.when(sid == 0)
            def _():
                pltpu.sync_copy(z2_ref.at[pl.ds(0, TR), pl.ds(0, DE)],
                                acc_sh.at[pl.ds(CH, TR), :])
                pltpu.sync_copy(z1_ref.at[pl.ds(0, TR)],
                                cnt_sh.at[pl.ds(CH, TR)])

            plsc.subcore_barrier()

            fire_gather(0, 0)
            fire_gather(1, 1)

            def turn(kk, slot):
                buf, idx, idxp = bufs[slot], idxs[slot], idxps[slot]
                wait_gather(kk, slot)
                for r in range(NR):
                    for l in range(8):
                        v = idx[r, pl.ds(l * LANES, LANES)]
                        ok = (v >= lo) & (v < lo + size)
                        trash = CH + ((r * 8 + l) % 4) * LANES + iota16
                        idxp[r, pl.ds(l * LANES, LANES)] = jnp.where(
                            ok, v - lo, trash)
                descs = []
                for r in range(NR):
                    descs.append(pltpu.async_copy(
                        buf.at[pl.ds(r * 128, 128), :],
                        acc_sh.at[idxp.at[r]], ssem, add=True))
                    descs.append(pltpu.async_copy(
                        ones, cnt_sh.at[idxp.at[r]], ssem, add=True))
                for d in descs:
                    d.wait()

            def pair(k2, _):
                for slot in range(2):
                    kk = 2 * k2 + slot
                    turn(kk, slot)

                    @pl.when(kk + 2 < NB)
                    def _():
                        fire_gather(kk + 2, slot)
                return 0

            lax.fori_loop(0, NB // 2, pair, 0)
            turn(NB - 1, (NB - 1) % 2)
            plsc.subcore_barrier()

            nsub, dtail = divmod(sl, B)

            def dump_piece(r0, nrows):
                pltpu.sync_copy(acc_sh.at[pl.ds(r0, nrows), :],
                                buf0.at[pl.ds(0, nrows), :])
                pltpu.sync_copy(cnt_sh.at[pl.ds(r0, nrows)],
                                cbuf.at[pl.ds(0, nrows)])
                _scale_rows(buf0, cbuf, 0, nrows, DE)
                pltpu.sync_copy(buf0.at[pl.ds(0, nrows), :],
                                mean_out.at[cid, pl.ds(lo + r0, nrows), :])

            def dump(i, _):
                dump_piece(sid * sl + i * B, B)
                return 0

            lax.fori_loop(0, nsub, dump, 0)
            if dtail:
                dump_piece(sid * sl + nsub * B, dtail)

    return k(ea, ei3d, z2, z1)


def _finalize(mean, rows, dim, blk):
    def body(m_ref, out_ref):
        out_ref[...] = m_ref[0] + m_ref[1]

    assert rows % blk == 0
    return pl.pallas_call(
        body,
        out_shape=jax.ShapeDtypeStruct((rows, dim), jnp.float32),
        grid=(rows // blk,),
        in_specs=[pl.BlockSpec((2, blk, dim), lambda i: (0, i, 0))],
        out_specs=pl.BlockSpec((blk, dim), lambda i: (i, 0)),
    )(mean)


def kernel(x, lg_node_idx, edge_attr, edge_index, org_edge_attr, org_x,
           org_edge_index):
    N, D = org_x.shape
    E, DE = org_edge_attr.shape
    ELG = edge_attr.shape[0]

    lg2d = lg_node_idx.T.reshape(2, E // 100, 100)
    ei3d = edge_index.reshape(2, ELG // 128, 128)
    z2 = jnp.zeros((832, 128), jnp.float32)
    z1 = jnp.zeros((-(-E // (4 * TILES) // 8) * 8,), jnp.float32)

    nmean = _node_sc(x, lg2d, z2, z1, N, E, D)
    emean = _edge_sc(edge_attr, ei3d, z2, z1, E, ELG, DE)

    new_x = _finalize(nmean, N, D, 1000 if N % 1000 == 0 else N)
    WF = 128 // DE
    ew = _finalize(emean.reshape(2, E // WF, WF * DE), E // WF, WF * DE,
                   4000 if (E // WF) % 4000 == 0 else E // WF)
    new_edge_attr = ew.reshape(E, DE)
    return new_x, new_edge_attr
